# Initial kernel scaffold; baseline (speedup 1.0000x reference)
import jax
import jax.numpy as jnp
from jax.experimental import pallas as pl


def kernel(per_atom_property_tensor, per_molecule_values, atomic_subsystem_indices):
    raise NotImplementedError("write your pallas kernel here")


# speedup vs baseline: 2.1151x; 2.1151x over previous
import jax
import jax.numpy as jnp
from jax.experimental import pallas as pl

N = 100000
M = 1000
D = 128
BLK = 2000


def _concat_body(x_ref, v_ref, idx_ref, out_ref):
    idx = idx_ref[...]
    hi = idx >> 5
    lo = idx & 31
    iota = jax.lax.broadcasted_iota(jnp.int32, (BLK, 32), 1)
    onehot_hi = (iota == hi).astype(jnp.float32)
    rows = jnp.dot(onehot_hi, v_ref[...], preferred_element_type=jnp.float32)
    col = jnp.sum(jnp.where(iota == lo, rows, 0.0), axis=1, keepdims=True)
    out_ref[:, :D] = x_ref[...]
    out_ref[:, D:D + 1] = col


def kernel(per_atom_property_tensor, per_molecule_values, atomic_subsystem_indices):
    v2d = jnp.zeros((32, 32), jnp.float32).reshape(-1).at[:M].set(
        per_molecule_values).reshape(32, 32)
    idx2d = atomic_subsystem_indices.reshape(N, 1)
    return pl.pallas_call(
        _concat_body,
        grid=(N // BLK,),
        in_specs=[
            pl.BlockSpec((BLK, D), lambda i: (i, 0)),
            pl.BlockSpec((32, 32), lambda i: (0, 0)),
            pl.BlockSpec((BLK, 1), lambda i: (i, 0)),
        ],
        out_specs=pl.BlockSpec((BLK, D + 1), lambda i: (i, 0)),
        out_shape=jax.ShapeDtypeStruct((N, D + 1), jnp.float32),
    )(per_atom_property_tensor, v2d, idx2d)
